# baseline (speedup 1.0000x reference)
import jax
import jax.numpy as jnp
from jax.experimental import pallas as pl
from jax.experimental.pallas import tpu as pltpu

_N_CODES = 20
_REPEAT = 10
_TOTAL = _N_CODES * _REPEAT
_BLK = 32
_NBUF = 6


def _body(x_hbm, w_ref, o_hbm, xbuf, obuf, in_sems, out_sems):
    nblk = x_hbm.shape[0] // _BLK
    w = w_ref[...]
    rows = jax.lax.broadcasted_iota(jnp.int32, (_TOTAL, _N_CODES), 0) // _REPEAT
    cols = jax.lax.broadcasted_iota(jnp.int32, (_TOTAL, _N_CODES), 1)
    wexp = jax.lax.dot((rows == cols).astype(jnp.float32), w,
                       precision=jax.lax.Precision.HIGHEST)

    def in_copy(i, slot):
        return pltpu.make_async_copy(
            x_hbm.at[pl.ds(i * _BLK, _BLK)], xbuf.at[slot], in_sems.at[slot])

    def out_copy(i, slot):
        return pltpu.make_async_copy(
            obuf.at[slot], o_hbm.at[pl.ds(i * _BLK, _BLK)], out_sems.at[slot])

    for s in range(_NBUF):
        in_copy(s, s).start()

    def step(i, carry):
        slot = jax.lax.rem(i, _NBUF)
        in_copy(i, slot).wait()

        @pl.when(i >= _NBUF)
        def _():
            out_copy(i - _NBUF, slot).wait()

        obuf[slot] = xbuf[slot] + wexp[None]
        out_copy(i, slot).start()

        @pl.when(i + _NBUF < nblk)
        def _():
            in_copy(i + _NBUF, slot).start()

        return carry

    jax.lax.fori_loop(0, nblk, step, 0)

    for i in range(nblk - _NBUF, nblk):
        out_copy(i, i % _NBUF).wait()


def kernel(X, W):
    B, T, D = X.shape
    return pl.pallas_call(
        _body,
        in_specs=[
            pl.BlockSpec(memory_space=pltpu.MemorySpace.HBM),
            pl.BlockSpec(memory_space=pltpu.MemorySpace.VMEM),
        ],
        out_specs=pl.BlockSpec(memory_space=pltpu.MemorySpace.HBM),
        out_shape=jax.ShapeDtypeStruct(X.shape, X.dtype),
        scratch_shapes=[
            pltpu.VMEM((_NBUF, _BLK, T, D), jnp.float32),
            pltpu.VMEM((_NBUF, _BLK, T, D), jnp.float32),
            pltpu.SemaphoreType.DMA((_NBUF,)),
            pltpu.SemaphoreType.DMA((_NBUF,)),
        ],
    )(X, W)


# speedup vs baseline: 6.3678x; 6.3678x over previous
import jax
import jax.numpy as jnp
from jax.experimental import pallas as pl

_N_CODES = 20
_REPEAT = 10


def _body(x_ref, wt_ref, o_ref):
    i = pl.program_id(0)
    wt = wt_ref[...]
    mask = (jax.lax.broadcasted_iota(jnp.int32, wt.shape, 1) == i)
    wcol = jnp.sum(jnp.where(mask, wt, 0.0), axis=1, keepdims=True)
    o_ref[...] = x_ref[...] + wcol[None, :, :]


def kernel(X, W):
    B, T, D = X.shape
    Xt = jnp.transpose(X, (1, 2, 0))
    Wt = jnp.transpose(W)
    out_t = pl.pallas_call(
        _body,
        grid=(_N_CODES,),
        in_specs=[
            pl.BlockSpec((_REPEAT, D, B), lambda i: (i, 0, 0)),
            pl.BlockSpec((D, _N_CODES), lambda i: (0, 0)),
        ],
        out_specs=pl.BlockSpec((_REPEAT, D, B), lambda i: (i, 0, 0)),
        out_shape=jax.ShapeDtypeStruct((T, D, B), X.dtype),
    )(Xt, Wt)
    return jnp.transpose(out_t, (2, 0, 1))
